# Initial kernel scaffold; baseline (speedup 1.0000x reference)
import jax
import jax.numpy as jnp
from jax.experimental import pallas as pl


def kernel(input_ids, position_ids, token_type_ids, inputs_embeds, triletter_table, position_table):
    raise NotImplementedError("write your pallas kernel here")


# speedup vs baseline: 12.4034x; 12.4034x over previous
import functools

import jax
import jax.numpy as jnp
from jax import lax
from jax.experimental import pallas as pl
from jax.experimental.pallas import tpu as pltpu
from jax.experimental.pallas import tpu_sc as plsc

VOCAB = 100000
HIDDEN = 64
MAXPOS = 512
TRI = 20
B = 4096
SEQ = 20

NC = 2
NS = 16
NW = NC * NS
NSEG = B * SEQ
SEG_PER_W = NSEG // NW
CS = 64
NCHUNK = SEG_PER_W // CS


def _body(ids_hbm, pos_hbm, tri_hbm, post_hbm, out_hbm,
          idx_v, pidx_v, rows_v, posr_v, out_v, sem):
    wid = lax.axis_index("s") * NC + lax.axis_index("c")
    wbase = wid * SEG_PER_W

    def chunk_body(c, _):
        seg0 = wbase + c * CS
        pltpu.sync_copy(ids_hbm.at[pl.ds(seg0 * TRI, CS * TRI)], idx_v)
        pltpu.sync_copy(pos_hbm.at[pl.ds(seg0, CS)], pidx_v)
        g1 = pltpu.async_copy(tri_hbm.at[idx_v], rows_v, sem)
        g2 = pltpu.async_copy(post_hbm.at[pidx_v], posr_v, sem)
        g1.wait()
        g2.wait()

        def seg_body(s, _):
            base = s * TRI
            for h in range(HIDDEN // 16):
                col = pl.ds(h * 16, 16)
                acc = posr_v[s, col]
                for t in range(TRI):
                    acc = acc + rows_v[base + t, col]
                out_v[s, col] = acc
            return _

        lax.fori_loop(0, CS, seg_body, None)
        pltpu.sync_copy(out_v, out_hbm.at[pl.ds(seg0, CS)])
        return _

    lax.fori_loop(0, NCHUNK, chunk_body, None)


@jax.jit
def _run(ids_flat, pos_flat, tri_table, pos_table):
    mesh = plsc.VectorSubcoreMesh(core_axis_name="c", subcore_axis_name="s",
                                  num_cores=NC, num_subcores=NS)
    f = pl.kernel(
        _body,
        out_type=jax.ShapeDtypeStruct((NSEG, HIDDEN), jnp.float32),
        mesh=mesh,
        scratch_types=[
            pltpu.VMEM((CS * TRI,), jnp.int32),
            pltpu.VMEM((CS,), jnp.int32),
            pltpu.VMEM((CS * TRI, HIDDEN), jnp.float32),
            pltpu.VMEM((CS, HIDDEN), jnp.float32),
            pltpu.VMEM((CS, HIDDEN), jnp.float32),
            pltpu.SemaphoreType.DMA,
        ],
        compiler_params=pltpu.CompilerParams(use_tc_tiling_on_sc=False),
    )
    return f(ids_flat, pos_flat, tri_table, pos_table)


def kernel(input_ids, position_ids, token_type_ids, inputs_embeds,
           triletter_table, position_table):
    ids_flat = input_ids.reshape(-1).astype(jnp.int32)
    pos_flat = position_ids.reshape(-1).astype(jnp.int32)
    out = _run(ids_flat, pos_flat, triletter_table, position_table)
    return out.reshape(B, SEQ, HIDDEN)


# speedup vs baseline: 15.9709x; 1.2876x over previous
import functools

import jax
import jax.numpy as jnp
from jax import lax
from jax.experimental import pallas as pl
from jax.experimental.pallas import tpu as pltpu
from jax.experimental.pallas import tpu_sc as plsc

VOCAB = 100000
HIDDEN = 64
MAXPOS = 512
TRI = 20
B = 4096
SEQ = 20

NC = 2
NS = 16
NW = NC * NS
NSEG = B * SEQ
SEG_PER_W = NSEG // NW
CS = 512
NCHUNK = SEG_PER_W // CS


def _body(ids_hbm, pos_hbm, tri_hbm, post_hbm, out_hbm,
          idx_v, pidx_v, acc_v, sem, psem):
    wid = lax.axis_index("s") * NC + lax.axis_index("c")
    wbase = wid * SEG_PER_W

    def chunk_body(c, _):
        seg0 = wbase + c * CS
        pltpu.sync_copy(pos_hbm.at[pl.ds(seg0, CS)], pidx_v)
        pltpu.async_copy(post_hbm.at[pidx_v], acc_v, psem).wait()
        for t in range(TRI):
            pltpu.sync_copy(ids_hbm.at[t, pl.ds(seg0, CS)], idx_v)
            pltpu.async_copy(tri_hbm.at[idx_v], acc_v, sem, add=True).wait()
        pltpu.sync_copy(acc_v, out_hbm.at[pl.ds(seg0, CS)])
        return _

    lax.fori_loop(0, NCHUNK, chunk_body, None)


@jax.jit
def _run(ids_t, pos_flat, tri_table, pos_table):
    mesh = plsc.VectorSubcoreMesh(core_axis_name="c", subcore_axis_name="s",
                                  num_cores=NC, num_subcores=NS)
    f = pl.kernel(
        _body,
        out_type=jax.ShapeDtypeStruct((NSEG, HIDDEN), jnp.float32),
        mesh=mesh,
        scratch_types=[
            pltpu.VMEM((CS,), jnp.int32),
            pltpu.VMEM((CS,), jnp.int32),
            pltpu.VMEM((CS, HIDDEN), jnp.float32),
            pltpu.SemaphoreType.DMA,
            pltpu.SemaphoreType.DMA,
        ],
        compiler_params=pltpu.CompilerParams(use_tc_tiling_on_sc=False),
    )
    return f(ids_t, pos_flat, tri_table, pos_table)


def kernel(input_ids, position_ids, token_type_ids, inputs_embeds,
           triletter_table, position_table):
    ids_t = input_ids.reshape(NSEG, TRI).T.reshape(TRI, NSEG)
    ids_t = jnp.asarray(ids_t, jnp.int32)
    pos_flat = position_ids.reshape(-1).astype(jnp.int32)
    out = _run(ids_t, pos_flat, triletter_table, position_table)
    return out.reshape(B, SEQ, HIDDEN)


# speedup vs baseline: 20.0668x; 1.2565x over previous
import functools

import jax
import jax.numpy as jnp
from jax import lax
from jax.experimental import pallas as pl
from jax.experimental.pallas import tpu as pltpu
from jax.experimental.pallas import tpu_sc as plsc

VOCAB = 100000
HIDDEN = 64
MAXPOS = 512
TRI = 20
B = 4096
SEQ = 20

NC = 2
NS = 16
NW = NC * NS
NSEG = B * SEQ
SEG_PER_W = NSEG // NW
CS = 256
NPAIR = SEG_PER_W // (2 * CS)


def _transpose_ids(ids_v, idxT_v):
    iota = lax.iota(jnp.int32, 16)
    for t in range(TRI):
        for g in range(CS // 16):
            src = t + TRI * (g * 16 + iota)
            idxT_v[t, pl.ds(g * 16, 16)] = plsc.load_gather(ids_v, [src])


def _body(ids_hbm, pos_hbm, tri_hbm, post_hbm, out_hbm,
          idsA_v, idsB_v, idxTA_v, idxTB_v, accA_v, accB_v,
          pidxA_v, pidxB_v, semA, semB, psemA, psemB):
    wid = lax.axis_index("s") * NC + lax.axis_index("c")
    wbase = wid * SEG_PER_W

    def pair_body(p, _):
        segA = wbase + (2 * p) * CS
        segB = segA + CS
        pltpu.sync_copy(ids_hbm.at[pl.ds(segA * TRI, CS * TRI)], idsA_v)
        pltpu.sync_copy(ids_hbm.at[pl.ds(segB * TRI, CS * TRI)], idsB_v)
        pltpu.sync_copy(pos_hbm.at[pl.ds(segA, CS)], pidxA_v)
        pltpu.sync_copy(pos_hbm.at[pl.ds(segB, CS)], pidxB_v)
        gA = pltpu.async_copy(post_hbm.at[pidxA_v], accA_v, psemA)
        gB = pltpu.async_copy(post_hbm.at[pidxB_v], accB_v, psemB)
        _transpose_ids(idsA_v, idxTA_v)
        _transpose_ids(idsB_v, idxTB_v)
        gA.wait()
        gB.wait()

        def add_body(t, _):
            dA = pltpu.async_copy(tri_hbm.at[idxTA_v.at[t]], accA_v, semA,
                                  add=True)
            dB = pltpu.async_copy(tri_hbm.at[idxTB_v.at[t]], accB_v, semB,
                                  add=True)
            dA.wait()
            dB.wait()
            return _

        lax.fori_loop(0, TRI, add_body, None)
        pltpu.sync_copy(accA_v, out_hbm.at[pl.ds(segA, CS)])
        pltpu.sync_copy(accB_v, out_hbm.at[pl.ds(segB, CS)])
        return _

    lax.fori_loop(0, NPAIR, pair_body, None)


@jax.jit
def _run(ids_flat, pos_flat, tri_table, pos_table):
    mesh = plsc.VectorSubcoreMesh(core_axis_name="c", subcore_axis_name="s",
                                  num_cores=NC, num_subcores=NS)
    f = pl.kernel(
        _body,
        out_type=jax.ShapeDtypeStruct((NSEG, HIDDEN), jnp.float32),
        mesh=mesh,
        scratch_types=[
            pltpu.VMEM((CS * TRI,), jnp.int32),
            pltpu.VMEM((CS * TRI,), jnp.int32),
            pltpu.VMEM((TRI, CS), jnp.int32),
            pltpu.VMEM((TRI, CS), jnp.int32),
            pltpu.VMEM((CS, HIDDEN), jnp.float32),
            pltpu.VMEM((CS, HIDDEN), jnp.float32),
            pltpu.VMEM((CS,), jnp.int32),
            pltpu.VMEM((CS,), jnp.int32),
            pltpu.SemaphoreType.DMA,
            pltpu.SemaphoreType.DMA,
            pltpu.SemaphoreType.DMA,
            pltpu.SemaphoreType.DMA,
        ],
        compiler_params=pltpu.CompilerParams(use_tc_tiling_on_sc=False,
                                             needs_layout_passes=False),
    )
    return f(ids_flat, pos_flat, tri_table, pos_table)


def kernel(input_ids, position_ids, token_type_ids, inputs_embeds,
           triletter_table, position_table):
    ids_flat = input_ids.reshape(-1).astype(jnp.int32)
    pos_flat = position_ids.reshape(-1).astype(jnp.int32)
    out = _run(ids_flat, pos_flat, triletter_table, position_table)
    return out.reshape(B, SEQ, HIDDEN)


# speedup vs baseline: 23.3142x; 1.1618x over previous
import functools

import jax
import jax.numpy as jnp
from jax import lax
from jax.experimental import pallas as pl
from jax.experimental.pallas import tpu as pltpu
from jax.experimental.pallas import tpu_sc as plsc

VOCAB = 100000
HIDDEN = 64
MAXPOS = 512
TRI = 20
B = 4096
SEQ = 20

NC = 2
NS = 16
NW = NC * NS
NSEG = B * SEQ
SEG_PER_W = NSEG // NW
CS = 256
NPAIR = SEG_PER_W // (2 * CS)


def _transpose_ids(ids_v, idxT_v):
    iota = lax.iota(jnp.int32, 16)
    for t in range(TRI):
        for g in range(CS // 16):
            src = t + TRI * (g * 16 + iota)
            idxT_v[t, pl.ds(g * 16, 16)] = plsc.load_gather(ids_v, [src])


def _body(ids_hbm, pos_hbm, tri_hbm, post_hbm, out_hbm,
          idsA_v, idsB_v, idxTA_v, idxTB_v, accA_v, accB_v,
          pidxA_v, pidxB_v, semA, semB, psemA, psemB):
    wid = lax.axis_index("s") * NC + lax.axis_index("c")
    wbase = wid * SEG_PER_W

    def pair_body(p, _):
        segA = wbase + (2 * p) * CS
        segB = segA + CS
        pltpu.sync_copy(ids_hbm.at[pl.ds(segA * TRI, CS * TRI)], idsA_v)
        pltpu.sync_copy(ids_hbm.at[pl.ds(segB * TRI, CS * TRI)], idsB_v)
        pltpu.sync_copy(pos_hbm.at[pl.ds(segA, CS)], pidxA_v)
        pltpu.sync_copy(pos_hbm.at[pl.ds(segB, CS)], pidxB_v)
        gA = pltpu.async_copy(post_hbm.at[pidxA_v], accA_v, psemA)
        gB = pltpu.async_copy(post_hbm.at[pidxB_v], accB_v, psemB)
        _transpose_ids(idsA_v, idxTA_v)
        _transpose_ids(idsB_v, idxTB_v)
        gA.wait()
        gB.wait()

        descs = []
        for t in range(TRI):
            descs.append(pltpu.async_copy(tri_hbm.at[idxTA_v.at[t]], accA_v,
                                          semA, add=True))
            descs.append(pltpu.async_copy(tri_hbm.at[idxTB_v.at[t]], accB_v,
                                          semB, add=True))
        for d in descs:
            d.wait()
        pltpu.sync_copy(accA_v, out_hbm.at[pl.ds(segA, CS)])
        pltpu.sync_copy(accB_v, out_hbm.at[pl.ds(segB, CS)])
        return _

    lax.fori_loop(0, NPAIR, pair_body, None)


@jax.jit
def _run(ids_flat, pos_flat, tri_table, pos_table):
    mesh = plsc.VectorSubcoreMesh(core_axis_name="c", subcore_axis_name="s",
                                  num_cores=NC, num_subcores=NS)
    f = pl.kernel(
        _body,
        out_type=jax.ShapeDtypeStruct((NSEG, HIDDEN), jnp.float32),
        mesh=mesh,
        scratch_types=[
            pltpu.VMEM((CS * TRI,), jnp.int32),
            pltpu.VMEM((CS * TRI,), jnp.int32),
            pltpu.VMEM((TRI, CS), jnp.int32),
            pltpu.VMEM((TRI, CS), jnp.int32),
            pltpu.VMEM((CS, HIDDEN), jnp.float32),
            pltpu.VMEM((CS, HIDDEN), jnp.float32),
            pltpu.VMEM((CS,), jnp.int32),
            pltpu.VMEM((CS,), jnp.int32),
            pltpu.SemaphoreType.DMA,
            pltpu.SemaphoreType.DMA,
            pltpu.SemaphoreType.DMA,
            pltpu.SemaphoreType.DMA,
        ],
        compiler_params=pltpu.CompilerParams(use_tc_tiling_on_sc=False,
                                             needs_layout_passes=False),
    )
    return f(ids_flat, pos_flat, tri_table, pos_table)


def kernel(input_ids, position_ids, token_type_ids, inputs_embeds,
           triletter_table, position_table):
    ids_flat = input_ids.reshape(-1).astype(jnp.int32)
    pos_flat = position_ids.reshape(-1).astype(jnp.int32)
    out = _run(ids_flat, pos_flat, triletter_table, position_table)
    return out.reshape(B, SEQ, HIDDEN)


# speedup vs baseline: 24.7574x; 1.0619x over previous
import functools

import jax
import jax.numpy as jnp
from jax import lax
from jax.experimental import pallas as pl
from jax.experimental.pallas import tpu as pltpu
from jax.experimental.pallas import tpu_sc as plsc

VOCAB = 100000
HIDDEN = 64
MAXPOS = 512
TRI = 20
B = 4096
SEQ = 20

NC = 2
NS = 16
NW = NC * NS
NSEG = B * SEQ
SEG_PER_W = NSEG // NW
CS = 512
NCHUNK = SEG_PER_W // CS


def _body(ids_hbm, pos_hbm, tri_hbm, post_hbm, out_hbm,
          ids2_v, idxT2_v, acc2_v, pidx2_v, addsem, psem, outsem):
    wid = lax.axis_index("s") * NC + lax.axis_index("c")
    wbase = wid * SEG_PER_W
    iota = lax.iota(jnp.int32, 16)

    def stage(c, buf):
        seg0 = wbase + c * CS
        pltpu.sync_copy(ids_hbm.at[pl.ds(seg0 * TRI, CS * TRI)],
                        ids2_v.at[buf])
        pltpu.sync_copy(pos_hbm.at[pl.ds(seg0, CS)], pidx2_v.at[buf])
        pltpu.async_copy(post_hbm.at[pidx2_v.at[buf]], acc2_v.at[buf], psem)

        def tr_body(g, _):
            for t in range(TRI):
                src = t + TRI * (g * 16 + iota)
                vec = plsc.load_gather(ids2_v.at[buf], [src])
                idxT2_v[buf, t, pl.ds(g * 16, 16)] = vec
            return _

        lax.fori_loop(0, CS // 16, tr_body, None)

    stage(0, 0)

    def chunk_body(c, _):
        cur = lax.rem(c, 2)
        nxt = lax.rem(c + 1, 2)
        seg0 = wbase + c * CS
        pltpu.make_async_copy(post_hbm.at[pidx2_v.at[cur]], acc2_v.at[cur],
                              psem).wait()
        descs = [
            pltpu.async_copy(tri_hbm.at[idxT2_v.at[cur, t]], acc2_v.at[cur],
                             addsem, add=True)
            for t in range(TRI)
        ]

        @pl.when(c > 0)
        def _():
            pltpu.make_async_copy(acc2_v.at[nxt],
                                  out_hbm.at[pl.ds(seg0, CS)], outsem).wait()

        @pl.when(c < NCHUNK - 1)
        def _():
            stage(c + 1, nxt)

        for d in descs:
            d.wait()
        pltpu.async_copy(acc2_v.at[cur], out_hbm.at[pl.ds(seg0, CS)], outsem)
        return _

    lax.fori_loop(0, NCHUNK, chunk_body, None)
    last = lax.rem(NCHUNK - 1, 2)
    pltpu.make_async_copy(acc2_v.at[last],
                          out_hbm.at[pl.ds(wbase, CS)], outsem).wait()


@jax.jit
def _run(ids_flat, pos_flat, tri_table, pos_table):
    mesh = plsc.VectorSubcoreMesh(core_axis_name="c", subcore_axis_name="s",
                                  num_cores=NC, num_subcores=NS)
    f = pl.kernel(
        _body,
        out_type=jax.ShapeDtypeStruct((NSEG, HIDDEN), jnp.float32),
        mesh=mesh,
        scratch_types=[
            pltpu.VMEM((2, CS * TRI), jnp.int32),
            pltpu.VMEM((2, TRI, CS), jnp.int32),
            pltpu.VMEM((2, CS, HIDDEN), jnp.float32),
            pltpu.VMEM((2, CS), jnp.int32),
            pltpu.SemaphoreType.DMA,
            pltpu.SemaphoreType.DMA,
            pltpu.SemaphoreType.DMA,
        ],
        compiler_params=pltpu.CompilerParams(use_tc_tiling_on_sc=False,
                                             needs_layout_passes=False),
    )
    return f(ids_flat, pos_flat, tri_table, pos_table)


def kernel(input_ids, position_ids, token_type_ids, inputs_embeds,
           triletter_table, position_table):
    ids_flat = input_ids.reshape(-1).astype(jnp.int32)
    pos_flat = position_ids.reshape(-1).astype(jnp.int32)
    out = _run(ids_flat, pos_flat, triletter_table, position_table)
    return out.reshape(B, SEQ, HIDDEN)
